# baseline (speedup 1.0000x reference)
import functools

import jax
import jax.numpy as jnp
from jax import lax
from jax.experimental import pallas as pl
from jax.experimental.pallas import tpu as pltpu
from jax.experimental.pallas import tpu_sc as plsc

_LANES = 16
_CHUNK = 128
_G = 4
_P = _CHUNK // _G


@functools.cache
def _build(B, T, V, D):
    info = plsc.get_sparse_core_info()
    NC, NS = info.num_cores, info.num_subcores
    NW = NC * NS
    FLAT = B * T
    assert B % NW == 0 and D % _LANES == 0
    seq_per_w = B // NW
    per_w = FLAT // NW
    assert seq_per_w % _G == 0 and T % _P == 0
    sg_count = seq_per_w // _G
    pb_count = T // _P
    n_chunks = sg_count * pb_count
    assert n_chunks % 2 == 0
    mesh = plsc.VectorSubcoreMesh(core_axis_name="c", subcore_axis_name="s")

    @functools.partial(
        pl.kernel,
        mesh=mesh,
        out_type=jax.ShapeDtypeStruct((FLAT, D), jnp.float32),
        scratch_types=[
            pltpu.VMEM((n_chunks, _CHUNK), jnp.int32),
            pltpu.VMEM((T, D), jnp.float32),
            pltpu.VMEM((_CHUNK, D), jnp.float32),
            pltpu.VMEM((_CHUNK, D), jnp.float32),
            pltpu.SemaphoreType.DMA,
            pltpu.SemaphoreType.DMA,
            pltpu.SemaphoreType.DMA,
            pltpu.SemaphoreType.DMA,
        ],
    )
    def k(tok_hbm, table_hbm, pos_hbm, out_hbm, idx_v, pos_v,
          rows0, rows1, g0, g1, s0, s1):
        rows = (rows0, rows1)
        gsem = (g0, g1)
        ssem = (s0, s1)
        wid = lax.axis_index("s") * NC + lax.axis_index("c")
        pltpu.sync_copy(tok_hbm.at[wid], idx_v)
        pltpu.sync_copy(pos_hbm, pos_v)
        base = wid * per_w

        def add_pos(rows_v, c):
            p0 = lax.rem(c, pb_count) * _P

            @plsc.parallel_loop(0, _P, 1, unroll=2)
            def add_p(p):
                for j in range(D // _LANES):
                    sl = pl.ds(j * _LANES, _LANES)
                    pv = pos_v[p0 + p, sl]
                    for g in range(_G):
                        r = g * _P + p
                        rows_v[r, sl] = rows_v[r, sl] + pv

        def store_chunk(rows_v, c, sem):
            sg = lax.div(c, pb_count)
            p0 = lax.rem(c, pb_count) * _P
            for g in range(_G):
                row0 = base + (sg * _G + g) * T + p0
                pltpu.async_copy(
                    rows_v.at[pl.ds(g * _P, _P)],
                    out_hbm.at[pl.ds(row0, _P)], sem)

        def wait_store(rows_v, sem):
            pltpu.make_async_copy(
                rows_v, out_hbm.at[pl.ds(0, _CHUNK)], sem).wait()

        pltpu.async_copy(table_hbm.at[idx_v.at[0]], rows0, g0)

        def loop_body(g, carry):
            for b in (0, 1):
                c = 2 * g + b
                pltpu.make_async_copy(
                    table_hbm.at[idx_v.at[c]], rows[b], gsem[b]).wait()
                if b == 0:
                    @pl.when(g > 0)
                    def _():
                        wait_store(rows[1], ssem[1])
                    pltpu.async_copy(
                        table_hbm.at[idx_v.at[c + 1]], rows[1], gsem[1])
                else:
                    wait_store(rows[0], ssem[0])

                    @pl.when(g < n_chunks // 2 - 1)
                    def _():
                        pltpu.async_copy(
                            table_hbm.at[idx_v.at[c + 1]], rows[0], gsem[0])
                add_pos(rows[b], c)
                store_chunk(rows[b], c, ssem[b])
            return carry

        lax.fori_loop(0, n_chunks // 2, loop_body, 0)
        wait_store(rows1, s1)

    return k


def kernel(token_ids, word_embeddings, positional_embeddings):
    B, T = token_ids.shape
    V, D = word_embeddings.shape
    k = _build(B, T, V, D)
    info = plsc.get_sparse_core_info()
    NW = info.num_cores * info.num_subcores
    seq_per_w = B // NW
    sg_count = seq_per_w // _G
    pb_count = T // _P
    tok = (token_ids.astype(jnp.int32)
           .reshape(NW, sg_count, _G, pb_count, _P)
           .transpose(0, 1, 3, 2, 4)
           .reshape(NW, sg_count * pb_count, _CHUNK))
    out = k(tok, word_embeddings, positional_embeddings)
    return out.reshape(B, T, D)


# speedup vs baseline: 1.1997x; 1.1997x over previous
import functools

import jax
import jax.numpy as jnp
from jax import lax
from jax.experimental import pallas as pl
from jax.experimental.pallas import tpu as pltpu
from jax.experimental.pallas import tpu_sc as plsc

_LANES = 16
_CHUNK = 128
_NBUF = 4
_LOOK = 2


@functools.cache
def _build(B, T, V, D):
    info = plsc.get_sparse_core_info()
    NC, NS = info.num_cores, info.num_subcores
    NW = NC * NS
    FLAT = B * T
    assert T % NW == 0 and D % _LANES == 0
    P = T // NW
    G = _CHUNK // P
    assert B % G == 0
    n_chunks = B // G
    assert n_chunks % _NBUF == 0 and n_chunks >= 2 * _NBUF
    mesh = plsc.VectorSubcoreMesh(core_axis_name="c", subcore_axis_name="s")

    @functools.partial(
        pl.kernel,
        mesh=mesh,
        out_type=jax.ShapeDtypeStruct((FLAT, D), jnp.float32),
        scratch_types=(
            [
                pltpu.VMEM((n_chunks, _CHUNK), jnp.int32),
                pltpu.VMEM((P, D), jnp.float32),
            ]
            + [pltpu.VMEM((_CHUNK, D), jnp.float32) for _ in range(_NBUF)]
            + [pltpu.SemaphoreType.DMA for _ in range(2 * _NBUF)]
        ),
    )
    def k(tok_hbm, table_hbm, pos_hbm, out_hbm, idx_v, pos_v, *bufs_and_sems):
        rows = bufs_and_sems[:_NBUF]
        gsem = bufs_and_sems[_NBUF:2 * _NBUF]
        ssem = bufs_and_sems[2 * _NBUF:]
        wid = lax.axis_index("s") * NC + lax.axis_index("c")
        pltpu.sync_copy(tok_hbm.at[wid], idx_v)
        pltpu.sync_copy(pos_hbm.at[pl.ds(wid * P, P)], pos_v)

        def start_gather(c, b):
            pltpu.async_copy(table_hbm.at[idx_v.at[c]], rows[b], gsem[b])

        def wait_gather(c, b):
            pltpu.make_async_copy(
                table_hbm.at[idx_v.at[c]], rows[b], gsem[b]).wait()

        def add_pos(rows_v, c):
            @plsc.parallel_loop(0, P, 1, unroll=2)
            def add_p(p):
                for j in range(D // _LANES):
                    sl = pl.ds(j * _LANES, _LANES)
                    pv = pos_v[p, sl]
                    for g in range(G):
                        r = g * P + p
                        rows_v[r, sl] = rows_v[r, sl] + pv

        def store_chunk(rows_v, c, sem):
            for g in range(G):
                row0 = (c * G + g) * T + wid * P
                pltpu.async_copy(
                    rows_v.at[pl.ds(g * P, P)],
                    out_hbm.at[pl.ds(row0, P)], sem)

        def wait_store(rows_v, sem):
            pltpu.make_async_copy(
                rows_v, out_hbm.at[pl.ds(0, _CHUNK)], sem).wait()

        niter = n_chunks // _NBUF
        for c0 in range(_LOOK):
            start_gather(c0, c0)

        def loop_body(i, carry):
            for b0 in range(_NBUF):
                c = _NBUF * i + b0
                wait_gather(c, b0)
                fb = (b0 + _LOOK) % _NBUF
                if b0 >= _NBUF - _LOOK:
                    wait_store(rows[fb], ssem[fb])
                else:
                    @pl.when(i > 0)
                    def _():
                        wait_store(rows[fb], ssem[fb])
                if b0 < _NBUF - _LOOK:
                    start_gather(c + _LOOK, fb)
                else:
                    @pl.when(i < niter - 1)
                    def _():
                        start_gather(c + _LOOK, fb)
                add_pos(rows[b0], c)
                store_chunk(rows[b0], c, ssem[b0])
            return carry

        lax.fori_loop(0, niter, loop_body, 0)
        for c in range(n_chunks - (_NBUF - _LOOK), n_chunks):
            wait_store(rows[c % _NBUF], ssem[c % _NBUF])

    return k


def kernel(token_ids, word_embeddings, positional_embeddings):
    B, T = token_ids.shape
    V, D = word_embeddings.shape
    k = _build(B, T, V, D)
    info = plsc.get_sparse_core_info()
    NW = info.num_cores * info.num_subcores
    P = T // NW
    G = _CHUNK // P
    n_chunks = B // G
    tok = (token_ids.astype(jnp.int32)
           .reshape(n_chunks, G, NW, P)
           .transpose(2, 0, 1, 3)
           .reshape(NW, n_chunks, _CHUNK))
    out = k(tok, word_embeddings, positional_embeddings)
    return out.reshape(B, T, D)
